# Initial kernel scaffold; baseline (speedup 1.0000x reference)
import jax
import jax.numpy as jnp
from jax.experimental import pallas as pl


def kernel(user, candidate, user_soc_edge_index, user_sim_edge_index, item_sim_edge_index, item_users, user_items, friends_items, friends_lens, user_embedding, item_embedding, gnn_w_self, gnn_w_neigh, att_wq, att_wk, att_v, pred_w1, pred_b1, pred_w2, pred_b2):
    raise NotImplementedError("write your pallas kernel here")


# speedup vs baseline: 2.7955x; 2.7955x over previous
import functools

import jax
import jax.numpy as jnp
from jax import lax
from jax.experimental import pallas as pl
from jax.experimental.pallas import tpu as pltpu
from jax.experimental.pallas import tpu_sc as plsc

NCORE = 2
NSUB = 16
NWORKER = NCORE * NSUB

_D = 128
_ALL_D = 384
_NPAD = 10240
_E = 320000
_B = 1024
_HIST = 20
_NF = 10
_ATT = 64
_SDIM = 48

_HIGH = jax.lax.Precision.HIGHEST


def _leaky(x):
  return jnp.where(x >= 0, x, 0.01 * x)


@functools.lru_cache(maxsize=None)
def _make_segsum(n_pad, d, e):
  e_w = e // NWORKER
  ch = 80
  n_ch = e_w // ch
  rows_tile = n_pad // NSUB

  mesh = plsc.VectorSubcoreMesh(core_axis_name="c", subcore_axis_name="s")
  out_type = jax.ShapeDtypeStruct((NCORE, n_pad, d), jnp.float32)
  scratch = [
      pltpu.VMEM((ch,), jnp.int32),
      pltpu.VMEM((ch,), jnp.int32),
      pltpu.VMEM((ch, d), jnp.float32),
      pltpu.VMEM_SHARED((n_pad, d), jnp.float32),
      pltpu.SemaphoreType.DMA,
  ]

  def body(x_hbm, src_hbm, dst_hbm, z_hbm, *refs):
    (out_hbm, src_v, dst_v, rows_v, acc_sh, sem) = refs
    c = lax.axis_index("c")
    s = lax.axis_index("s")
    w = s * NCORE + c
    tb = s * rows_tile

    pltpu.sync_copy(z_hbm, acc_sh.at[pl.ds(tb, rows_tile), :])
    plsc.subcore_barrier()

    base = w * e_w

    def step(i, carry):
      off = base + i * ch
      pltpu.sync_copy(src_hbm.at[pl.ds(off, ch)], src_v)
      pltpu.sync_copy(dst_hbm.at[pl.ds(off, ch)], dst_v)
      pltpu.async_copy(x_hbm.at[src_v], rows_v, sem).wait()
      pltpu.sync_copy(rows_v, acc_sh.at[dst_v], add=True)
      return carry

    lax.fori_loop(0, n_ch, step, 0)
    plsc.subcore_barrier()
    pltpu.sync_copy(acc_sh.at[pl.ds(tb, rows_tile), :],
                    out_hbm.at[c, pl.ds(tb, rows_tile), :])

  return pl.kernel(body, out_type=out_type, mesh=mesh, scratch_types=scratch)


@functools.lru_cache(maxsize=None)
def _make_degree(n_pad, d, e):
  e_w = e // NWORKER
  ch = 80
  n_ch = e_w // ch
  rows_tile = n_pad // NSUB

  mesh = plsc.VectorSubcoreMesh(core_axis_name="c", subcore_axis_name="s")
  out_type = jax.ShapeDtypeStruct((NCORE, n_pad, d), jnp.float32)
  scratch = [
      pltpu.VMEM((ch,), jnp.int32),
      pltpu.VMEM((ch, d), jnp.float32),
      pltpu.VMEM_SHARED((n_pad, d), jnp.float32),
  ]

  def body(dst_hbm, z_hbm, ones_hbm, *refs):
    (deg_hbm, dst_v, ones_v, dacc_sh) = refs
    c = lax.axis_index("c")
    s = lax.axis_index("s")
    w = s * NCORE + c
    tb = s * rows_tile

    pltpu.sync_copy(z_hbm, dacc_sh.at[pl.ds(tb, rows_tile), :])
    pltpu.sync_copy(ones_hbm, ones_v)
    plsc.subcore_barrier()

    base = w * e_w

    def step(i, carry):
      off = base + i * ch
      pltpu.sync_copy(dst_hbm.at[pl.ds(off, ch)], dst_v)
      pltpu.sync_copy(ones_v, dacc_sh.at[dst_v], add=True)
      return carry

    lax.fori_loop(0, n_ch, step, 0)
    plsc.subcore_barrier()
    pltpu.sync_copy(dacc_sh.at[pl.ds(tb, rows_tile), :],
                    deg_hbm.at[c, pl.ds(tb, rows_tile), :])

  return pl.kernel(body, out_type=out_type, mesh=mesh, scratch_types=scratch)


@functools.lru_cache(maxsize=None)
def _make_fuse(n_pad, d_all, b, f, hist, emit_scale):
  nb = b // NWORKER
  pairs = nb * f
  pc = 4
  gi = pc * hist
  oc = 32 if f == 1 else 80
  cpw = oc // pc
  n_out = pairs // oc
  nv = d_all // 16

  mesh = plsc.VectorSubcoreMesh(core_axis_name="c", subcore_axis_name="s")
  if emit_scale:
    out_type = [jax.ShapeDtypeStruct((b * f, d_all), jnp.float32),
                jax.ShapeDtypeStruct((b, d_all), jnp.float32)]
  else:
    out_type = jax.ShapeDtypeStruct((b * f, d_all), jnp.float32)
  scratch = [
      pltpu.VMEM((nb,), jnp.int32),
      pltpu.VMEM((nb, d_all), jnp.float32),
      pltpu.VMEM((gi,), jnp.int32),
      pltpu.VMEM((gi, d_all), jnp.float32),
      pltpu.VMEM((oc, d_all), jnp.float32),
      pltpu.SemaphoreType.DMA,
  ]

  def body(table_hbm, sidx_hbm, hidx_hbm, *refs):
    if emit_scale:
      (fused_hbm, scale_hbm, sidx_v, scale_v, hidx_v, rows_v, out_v, sem) = refs
    else:
      (fused_hbm, sidx_v, scale_v, hidx_v, rows_v, out_v, sem) = refs
      scale_hbm = None
    c = lax.axis_index("c")
    s = lax.axis_index("s")
    w = s * NCORE + c

    pltpu.sync_copy(sidx_hbm.at[pl.ds(w * nb, nb)], sidx_v)
    pltpu.async_copy(table_hbm.at[sidx_v], scale_v, sem).wait()
    if emit_scale:
      pltpu.sync_copy(scale_v, scale_hbm.at[pl.ds(w * nb, nb), :])

    def outer(o, carry):
      def inner(t, carry2):
        chunk = o * cpw + t
        p0 = chunk * pc
        pltpu.sync_copy(hidx_hbm.at[pl.ds((w * pairs + p0) * hist, gi)],
                        hidx_v)
        pltpu.async_copy(table_hbm.at[hidx_v], rows_v, sem).wait()
        for j in range(pc):
          bloc = (p0 + j) // f
          for jv in range(nv):
            sl = pl.ds(jv * 16, 16)
            sc_vec = scale_v[bloc, sl]
            acc = rows_v[j * hist, sl] * sc_vec
            for h in range(1, hist):
              acc = jnp.maximum(acc, rows_v[j * hist + h, sl] * sc_vec)
            out_v[t * pc + j, sl] = acc
        return carry2

      lax.fori_loop(0, cpw, inner, 0)
      pltpu.sync_copy(out_v, fused_hbm.at[pl.ds(w * pairs + o * oc, oc), :])
      return carry

    lax.fori_loop(0, n_out, outer, 0)

  return pl.kernel(body, out_type=out_type, mesh=mesh, scratch_types=scratch)


def _layer_body(x_ref, p_ref, dp_ref, ws_ref, wn_ref, h_ref, hn_ref):
  x = x_ref[...]
  agg = p_ref[0] + p_ref[1]
  deg = dp_ref[0, :, 0:1] + dp_ref[1, :, 0:1]
  mean = agg / jnp.maximum(deg, 1.0)
  h = (jnp.dot(x, ws_ref[...], preferred_element_type=jnp.float32,
               precision=_HIGH)
       + jnp.dot(mean, wn_ref[...], preferred_element_type=jnp.float32,
                 precision=_HIGH))
  h = _leaky(h)
  nrm = jnp.sqrt(jnp.sum(h * h, axis=1, keepdims=True))
  hn = h / jnp.maximum(nrm, 1e-12)
  h_ref[...] = h
  hn_ref[...] = hn


def _layer_tc(x, parts, degp, ws, wn):
  blk = 2048
  grid = _NPAD // blk
  return pl.pallas_call(
      _layer_body,
      grid=(grid,),
      in_specs=[
          pl.BlockSpec((blk, _D), lambda i: (i, 0)),
          pl.BlockSpec((NCORE, blk, _D), lambda i: (0, i, 0)),
          pl.BlockSpec((NCORE, blk, _D), lambda i: (0, i, 0)),
          pl.BlockSpec((_D, _D), lambda i: (0, 0)),
          pl.BlockSpec((_D, _D), lambda i: (0, 0)),
      ],
      out_specs=[
          pl.BlockSpec((blk, _D), lambda i: (i, 0)),
          pl.BlockSpec((blk, _D), lambda i: (i, 0)),
      ],
      out_shape=[
          jax.ShapeDtypeStruct((_NPAD, _D), jnp.float32),
          jax.ShapeDtypeStruct((_NPAD, _D), jnp.float32),
      ],
  )(x, parts, degp, ws, wn)


def _assemble_body(ue_ref, s1_ref, s2_ref, m1_ref, m2_ref,
                   ie_ref, i1_ref, i2_ref, ug_ref, it_ref):
  i = pl.program_id(0)
  blk = ue_ref.shape[0]
  rows = jax.lax.broadcasted_iota(jnp.int32, (blk, 1), 0) + i * blk
  keep = rows != 0
  ug = jnp.concatenate(
      [ue_ref[...], 0.5 * (s1_ref[...] + m1_ref[...]),
       0.5 * (s2_ref[...] + m2_ref[...])], axis=1)
  it = jnp.concatenate([ie_ref[...], i1_ref[...], i2_ref[...]], axis=1)
  ug_ref[...] = jnp.where(keep, ug, 0.0)
  it_ref[...] = jnp.where(keep, it, 0.0)


def _assemble_tc(ue, soc1, soc2, sim1, sim2, ie, is1, is2):
  blk = 2048
  grid = _NPAD // blk
  spec_d = pl.BlockSpec((blk, _D), lambda i: (i, 0))
  spec_a = pl.BlockSpec((blk, _ALL_D), lambda i: (i, 0))
  return pl.pallas_call(
      _assemble_body,
      grid=(grid,),
      in_specs=[spec_d] * 8,
      out_specs=[spec_a, spec_a],
      out_shape=[
          jax.ShapeDtypeStruct((_NPAD, _ALL_D), jnp.float32),
          jax.ShapeDtypeStruct((_NPAD, _ALL_D), jnp.float32),
      ],
  )(ue, soc1, soc2, sim1, sim2, ie, is1, is2)


def _final_body(ug_ref, isg_ref, il_ref, usl_ref, femb_ref, lens_ref,
                wq_ref, wk_ref, av_ref, w1_ref, b1_ref, w2_ref, b2_ref,
                out_ref):
  blk = ug_ref.shape[0]
  usl = usl_ref[...]
  q = jnp.dot(usl, wq_ref[...], preferred_element_type=jnp.float32,
              precision=_HIGH)
  wk = wk_ref[...]
  av = av_ref[...]
  e_cols = []
  for f in range(_NF):
    kf = femb_ref[:, f, :]
    kp = jnp.dot(kf, wk, preferred_element_type=jnp.float32, precision=_HIGH)
    ef = jnp.dot(jnp.tanh(q + kp), av, preferred_element_type=jnp.float32,
                 precision=_HIGH)
    e_cols.append(ef)
  e = jnp.concatenate(e_cols, axis=1)
  lens = jnp.maximum(lens_ref[...], 1)
  mask = jax.lax.broadcasted_iota(jnp.int32, (blk, _NF), 1) < lens
  e = jnp.where(mask, e, -1e9)
  m = jnp.max(e, axis=1, keepdims=True)
  ex = jnp.exp(e - m)
  a = ex / jnp.sum(ex, axis=1, keepdims=True)
  ufl = jnp.zeros((blk, _ALL_D), jnp.float32)
  for f in range(_NF):
    ufl = ufl + a[:, f:f + 1] * femb_ref[:, f, :]
  ul = 0.5 * (usl + ufl)

  def predict(u, i, g):
    h = (jnp.dot(u, w1_ref[g, :_ALL_D, :], preferred_element_type=jnp.float32,
                 precision=_HIGH)
         + jnp.dot(i, w1_ref[g, _ALL_D:, :], preferred_element_type=jnp.float32,
                   precision=_HIGH)
         + b1_ref[g:g + 1, :])
    h = _leaky(h)
    return (jnp.dot(h, w2_ref[g], preferred_element_type=jnp.float32,
                    precision=_HIGH) + b2_ref[g:g + 1, :])

  ug = ug_ref[...]
  isg = isg_ref[...]
  s1 = predict(ug, isg, 0)
  s2 = predict(ul, isg, 1)
  s3 = predict(ug, il_ref[...], 2)
  out_ref[...] = (s1 + s2 + s3) * (1.0 / 3.0)


def _final_tc(ug, isg, il, usl, femb, lens2, att_wq, att_wk, att_v2,
              pred_w1, pred_b1, pred_w2, pred_b2):
  blk = 256
  grid = _B // blk
  spec_a = pl.BlockSpec((blk, _ALL_D), lambda i: (i, 0))
  return pl.pallas_call(
      _final_body,
      grid=(grid,),
      in_specs=[
          spec_a, spec_a, spec_a, spec_a,
          pl.BlockSpec((blk, _NF, _ALL_D), lambda i: (i, 0, 0)),
          pl.BlockSpec((blk, 1), lambda i: (i, 0)),
          pl.BlockSpec((_ALL_D, _ATT), lambda i: (0, 0)),
          pl.BlockSpec((_ALL_D, _ATT), lambda i: (0, 0)),
          pl.BlockSpec((_ATT, 1), lambda i: (0, 0)),
          pl.BlockSpec((3, 2 * _ALL_D, _SDIM), lambda i: (0, 0, 0)),
          pl.BlockSpec((3, _SDIM), lambda i: (0, 0)),
          pl.BlockSpec((3, _SDIM, 1), lambda i: (0, 0, 0)),
          pl.BlockSpec((3, 1), lambda i: (0, 0)),
      ],
      out_specs=pl.BlockSpec((blk, 1), lambda i: (i, 0)),
      out_shape=jax.ShapeDtypeStruct((_B, 1), jnp.float32),
  )(ug, isg, il, usl, femb, lens2, att_wq, att_wk, att_v2,
    pred_w1, pred_b1, pred_w2, pred_b2)


def _run_graph(x0, src, dst, ws, wn, zeros_zb, ones_ob):
  segsum = _make_segsum(_NPAD, _D, _E)
  degree = _make_degree(_NPAD, _D, _E)
  degp = degree(dst, zeros_zb, ones_ob)
  parts = segsum(x0, src, dst, zeros_zb)
  h1, n1 = _layer_tc(x0, parts, degp, ws[0], wn[0])
  parts2 = segsum(h1, src, dst, zeros_zb)
  _, n2 = _layer_tc(h1, parts2, degp, ws[1], wn[1])
  return n1, n2


def kernel(user, candidate, user_soc_edge_index, user_sim_edge_index,
           item_sim_edge_index, item_users, user_items, friends_items,
           friends_lens, user_embedding, item_embedding, gnn_w_self,
           gnn_w_neigh, att_wq, att_wk, att_v, pred_w1, pred_b1, pred_w2,
           pred_b2):
  f32 = jnp.float32
  i32 = jnp.int32
  n_u = user_embedding.shape[0]
  n_i = item_embedding.shape[0]

  ue = jnp.pad(user_embedding.astype(f32), ((0, _NPAD - n_u), (0, 0)))
  ie = jnp.pad(item_embedding.astype(f32), ((0, _NPAD - n_i), (0, 0)))
  zeros_zb = jnp.zeros((_NPAD // NSUB, _D), f32)
  ones_ob = jnp.ones((80, _D), f32)

  soc_src = user_soc_edge_index[0].astype(i32)
  soc_dst = user_soc_edge_index[1].astype(i32)
  sim_src = user_sim_edge_index[0].astype(i32)
  sim_dst = user_sim_edge_index[1].astype(i32)
  isim_src = item_sim_edge_index[0].astype(i32)
  isim_dst = item_sim_edge_index[1].astype(i32)

  soc1, soc2 = _run_graph(ue, soc_src, soc_dst, gnn_w_self[0],
                          gnn_w_neigh[0], zeros_zb, ones_ob)
  sim1, sim2 = _run_graph(ue, sim_src, sim_dst, gnn_w_self[1],
                          gnn_w_neigh[1], zeros_zb, ones_ob)
  is1, is2 = _run_graph(ie, isim_src, isim_dst, gnn_w_self[2],
                        gnn_w_neigh[2], zeros_zb, ones_ob)

  ug_table, it_table = _assemble_tc(ue, soc1, soc2, sim1, sim2, ie, is1, is2)

  fuse1 = _make_fuse(_NPAD, _ALL_D, _B, 1, _HIST, True)
  fuse_f = _make_fuse(_NPAD, _ALL_D, _B, _NF, _HIST, False)

  item_local, user_global = fuse1(ug_table, user.astype(i32),
                                  item_users.reshape(-1).astype(i32))
  user_sim_local, item_sim_global = fuse1(it_table, candidate.astype(i32),
                                          user_items.reshape(-1).astype(i32))
  f_emb = fuse_f(it_table, candidate.astype(i32),
                 friends_items.reshape(-1).astype(i32))

  femb = f_emb.reshape(_B, _NF, _ALL_D)
  lens2 = friends_lens.astype(i32).reshape(_B, 1)
  out = _final_tc(user_global, item_sim_global, item_local, user_sim_local,
                  femb, lens2, att_wq, att_wk, att_v.reshape(_ATT, 1),
                  pred_w1, pred_b1, pred_w2, pred_b2)
  return out.reshape(_B)


# speedup vs baseline: 4.6833x; 1.6753x over previous
import functools

import jax
import jax.numpy as jnp
from jax import lax
from jax.experimental import pallas as pl
from jax.experimental.pallas import tpu as pltpu
from jax.experimental.pallas import tpu_sc as plsc

NCORE = 2
NSUB = 16
NWORKER = NCORE * NSUB

_D = 128
_ALL_D = 384
_NPAD = 10240
_E = 320000
_B = 1024
_HIST = 20
_NF = 10
_ATT = 64
_SDIM = 48

_HIGH = jax.lax.Precision.HIGHEST


def _leaky(x):
  return jnp.where(x >= 0, x, 0.01 * x)


@functools.lru_cache(maxsize=None)
def _make_segsum(n_pad, d, e):
  e_w = e // NWORKER
  ch = 80
  n_ch = e_w // ch
  n2 = (n_ch - 1) // 2
  rows_tile = n_pad // NSUB

  mesh = plsc.VectorSubcoreMesh(core_axis_name="c", subcore_axis_name="s")
  out_type = jax.ShapeDtypeStruct((NCORE, n_pad, d), jnp.float32)
  scratch = [
      pltpu.VMEM((ch,), jnp.int32),
      pltpu.VMEM((ch,), jnp.int32),
      pltpu.VMEM((ch,), jnp.int32),
      pltpu.VMEM((ch,), jnp.int32),
      pltpu.VMEM((ch, d), jnp.float32),
      pltpu.VMEM((ch, d), jnp.float32),
      pltpu.VMEM_SHARED((n_pad, d), jnp.float32),
      pltpu.SemaphoreType.DMA,
      pltpu.SemaphoreType.DMA,
      pltpu.SemaphoreType.DMA,
      pltpu.SemaphoreType.DMA,
  ]

  def body(x_hbm, src_hbm, dst_hbm, z_hbm, *refs):
    (out_hbm, is0, is1, id0, id1, r0, r1, acc_sh, si0, si1, sg0, sg1) = refs
    c = lax.axis_index("c")
    s = lax.axis_index("s")
    w = s * NCORE + c
    tb = s * rows_tile

    pltpu.sync_copy(z_hbm, acc_sh.at[pl.ds(tb, rows_tile), :])
    plsc.subcore_barrier()

    base = w * e_w

    def fire_idx(ibs, ibd, sem, chunk):
      off = base + chunk * ch
      pltpu.async_copy(src_hbm.at[pl.ds(off, ch)], ibs, sem)
      pltpu.async_copy(dst_hbm.at[pl.ds(off, ch)], ibd, sem)

    def wait_idx(ibs, ibd, sem):
      pltpu.make_async_copy(src_hbm.at[pl.ds(0, ch)], ibs, sem).wait()
      pltpu.make_async_copy(dst_hbm.at[pl.ds(0, ch)], ibd, sem).wait()

    fire_idx(is0, id0, si0, 0)
    wait_idx(is0, id0, si0)
    pltpu.async_copy(x_hbm.at[is0], r0, sg0)
    fire_idx(is1, id1, si1, 1)

    def step(i, carry):
      c0 = 2 * i
      wait_idx(is1, id1, si1)
      pltpu.async_copy(x_hbm.at[is1], r1, sg1)
      pltpu.make_async_copy(x_hbm.at[is0], r0, sg0).wait()
      pltpu.sync_copy(r0, acc_sh.at[id0], add=True)
      fire_idx(is0, id0, si0, c0 + 2)
      wait_idx(is0, id0, si0)
      pltpu.async_copy(x_hbm.at[is0], r0, sg0)
      pltpu.make_async_copy(x_hbm.at[is1], r1, sg1).wait()
      pltpu.sync_copy(r1, acc_sh.at[id1], add=True)
      fire_idx(is1, id1, si1, c0 + 3)
      return carry

    lax.fori_loop(0, n2, step, 0)
    pltpu.make_async_copy(x_hbm.at[is0], r0, sg0).wait()
    pltpu.sync_copy(r0, acc_sh.at[id0], add=True)
    wait_idx(is1, id1, si1)

    plsc.subcore_barrier()
    pltpu.sync_copy(acc_sh.at[pl.ds(tb, rows_tile), :],
                    out_hbm.at[c, pl.ds(tb, rows_tile), :])

  return pl.kernel(body, out_type=out_type, mesh=mesh, scratch_types=scratch)


@functools.lru_cache(maxsize=None)
def _make_degree(n_pad, d, e):
  e_w = e // NWORKER
  ch = 80
  n_ch = e_w // ch
  n2 = (n_ch - 1) // 2
  rows_tile = n_pad // NSUB

  mesh = plsc.VectorSubcoreMesh(core_axis_name="c", subcore_axis_name="s")
  out_type = jax.ShapeDtypeStruct((NCORE, n_pad, d), jnp.float32)
  scratch = [
      pltpu.VMEM((ch,), jnp.int32),
      pltpu.VMEM((ch,), jnp.int32),
      pltpu.VMEM((ch, d), jnp.float32),
      pltpu.VMEM_SHARED((n_pad, d), jnp.float32),
      pltpu.SemaphoreType.DMA,
      pltpu.SemaphoreType.DMA,
  ]

  def body(dst_hbm, z_hbm, ones_hbm, *refs):
    (deg_hbm, id0, id1, ones_v, dacc_sh, si0, si1) = refs
    c = lax.axis_index("c")
    s = lax.axis_index("s")
    w = s * NCORE + c
    tb = s * rows_tile

    pltpu.sync_copy(z_hbm, dacc_sh.at[pl.ds(tb, rows_tile), :])
    pltpu.sync_copy(ones_hbm, ones_v)
    plsc.subcore_barrier()

    base = w * e_w

    def fire(ibd, sem, chunk):
      pltpu.async_copy(dst_hbm.at[pl.ds(base + chunk * ch, ch)], ibd, sem)

    def wait(ibd, sem):
      pltpu.make_async_copy(dst_hbm.at[pl.ds(0, ch)], ibd, sem).wait()

    fire(id0, si0, 0)
    fire(id1, si1, 1)

    def step(i, carry):
      c0 = 2 * i
      wait(id0, si0)
      pltpu.sync_copy(ones_v, dacc_sh.at[id0], add=True)
      fire(id0, si0, c0 + 2)
      wait(id1, si1)
      pltpu.sync_copy(ones_v, dacc_sh.at[id1], add=True)
      fire(id1, si1, c0 + 3)
      return carry

    lax.fori_loop(0, n2, step, 0)
    wait(id0, si0)
    pltpu.sync_copy(ones_v, dacc_sh.at[id0], add=True)
    wait(id1, si1)

    plsc.subcore_barrier()
    pltpu.sync_copy(dacc_sh.at[pl.ds(tb, rows_tile), :],
                    deg_hbm.at[c, pl.ds(tb, rows_tile), :])

  return pl.kernel(body, out_type=out_type, mesh=mesh, scratch_types=scratch)


@functools.lru_cache(maxsize=None)
def _make_fuse(n_pad, d_all, b, f, hist, emit_scale):
  nb = b // NWORKER
  pairs = nb * f
  pc = 4
  gi = pc * hist
  oc = 32 if f == 1 else 80
  cpw = oc // pc
  n_out = pairs // oc
  nv = d_all // 16

  mesh = plsc.VectorSubcoreMesh(core_axis_name="c", subcore_axis_name="s")
  if emit_scale:
    out_type = [jax.ShapeDtypeStruct((b * f, d_all), jnp.float32),
                jax.ShapeDtypeStruct((b, d_all), jnp.float32)]
  else:
    out_type = jax.ShapeDtypeStruct((b * f, d_all), jnp.float32)
  scratch = [
      pltpu.VMEM((nb,), jnp.int32),
      pltpu.VMEM((nb, d_all), jnp.float32),
      pltpu.VMEM((oc * hist,), jnp.int32),
      pltpu.VMEM((gi, d_all), jnp.float32),
      pltpu.VMEM((gi, d_all), jnp.float32),
      pltpu.VMEM((oc, d_all), jnp.float32),
      pltpu.SemaphoreType.DMA,
      pltpu.SemaphoreType.DMA,
      pltpu.SemaphoreType.DMA,
  ]
  c2 = cpw // 2

  def body(table_hbm, sidx_hbm, hidx_hbm, *refs):
    if emit_scale:
      (fused_hbm, scale_hbm, sidx_v, scale_v, hidx_v, r0, r1, out_v,
       sem, sg0, sg1) = refs
    else:
      (fused_hbm, sidx_v, scale_v, hidx_v, r0, r1, out_v,
       sem, sg0, sg1) = refs
      scale_hbm = None
    c = lax.axis_index("c")
    s = lax.axis_index("s")
    w = s * NCORE + c

    pltpu.sync_copy(sidx_hbm.at[pl.ds(w * nb, nb)], sidx_v)
    pltpu.async_copy(table_hbm.at[sidx_v], scale_v, sem).wait()
    if emit_scale:
      pltpu.sync_copy(scale_v, scale_hbm.at[pl.ds(w * nb, nb), :])

    def fire_g(rows, sg, t):
      tc = jnp.minimum(t, cpw - 1)
      pltpu.async_copy(table_hbm.at[hidx_v.at[pl.ds(tc * gi, gi)]], rows, sg)

    def wait_g(rows, sg):
      pltpu.make_async_copy(table_hbm.at[hidx_v.at[pl.ds(0, gi)]],
                            rows, sg).wait()

    def compute_chunk(rows, t, p_base):
      for j in range(pc):
        bloc = (p_base + j) // f
        for jv in range(nv):
          sl = pl.ds(jv * 16, 16)
          sc_vec = scale_v[bloc, sl]
          acc = rows[j * hist, sl] * sc_vec
          for h in range(1, hist):
            acc = jnp.maximum(acc, rows[j * hist + h, sl] * sc_vec)
          out_v[t * pc + j, sl] = acc

    def outer(o, carry):
      pltpu.sync_copy(hidx_hbm.at[pl.ds((w * pairs + o * oc) * hist,
                                        oc * hist)], hidx_v)
      fire_g(r0, sg0, 0)
      fire_g(r1, sg1, 1)

      def inner(i, carry2):
        ta = 2 * i
        p_blk = o * oc
        wait_g(r0, sg0)
        compute_chunk(r0, ta, p_blk + ta * pc)
        fire_g(r0, sg0, ta + 2)
        wait_g(r1, sg1)
        compute_chunk(r1, ta + 1, p_blk + (ta + 1) * pc)
        fire_g(r1, sg1, ta + 3)
        return carry2

      lax.fori_loop(0, c2, inner, 0)
      wait_g(r0, sg0)
      wait_g(r1, sg1)
      pltpu.sync_copy(out_v, fused_hbm.at[pl.ds(w * pairs + o * oc, oc), :])
      return carry

    lax.fori_loop(0, n_out, outer, 0)

  return pl.kernel(body, out_type=out_type, mesh=mesh, scratch_types=scratch)


def _layer_body(x_ref, p_ref, dp_ref, ws_ref, wn_ref, h_ref, hn_ref):
  x = x_ref[...]
  agg = p_ref[0] + p_ref[1]
  deg = dp_ref[0, :, 0:1] + dp_ref[1, :, 0:1]
  mean = agg / jnp.maximum(deg, 1.0)
  h = (jnp.dot(x, ws_ref[...], preferred_element_type=jnp.float32,
               precision=_HIGH)
       + jnp.dot(mean, wn_ref[...], preferred_element_type=jnp.float32,
                 precision=_HIGH))
  h = _leaky(h)
  nrm = jnp.sqrt(jnp.sum(h * h, axis=1, keepdims=True))
  hn = h / jnp.maximum(nrm, 1e-12)
  h_ref[...] = h
  hn_ref[...] = hn


def _layer_tc(x, parts, degp, ws, wn):
  blk = 2048
  grid = _NPAD // blk
  return pl.pallas_call(
      _layer_body,
      grid=(grid,),
      in_specs=[
          pl.BlockSpec((blk, _D), lambda i: (i, 0)),
          pl.BlockSpec((NCORE, blk, _D), lambda i: (0, i, 0)),
          pl.BlockSpec((NCORE, blk, _D), lambda i: (0, i, 0)),
          pl.BlockSpec((_D, _D), lambda i: (0, 0)),
          pl.BlockSpec((_D, _D), lambda i: (0, 0)),
      ],
      out_specs=[
          pl.BlockSpec((blk, _D), lambda i: (i, 0)),
          pl.BlockSpec((blk, _D), lambda i: (i, 0)),
      ],
      out_shape=[
          jax.ShapeDtypeStruct((_NPAD, _D), jnp.float32),
          jax.ShapeDtypeStruct((_NPAD, _D), jnp.float32),
      ],
  )(x, parts, degp, ws, wn)


def _assemble_body(ue_ref, s1_ref, s2_ref, m1_ref, m2_ref,
                   ie_ref, i1_ref, i2_ref, ug_ref, it_ref):
  i = pl.program_id(0)
  blk = ue_ref.shape[0]
  rows = jax.lax.broadcasted_iota(jnp.int32, (blk, 1), 0) + i * blk
  keep = rows != 0
  ug = jnp.concatenate(
      [ue_ref[...], 0.5 * (s1_ref[...] + m1_ref[...]),
       0.5 * (s2_ref[...] + m2_ref[...])], axis=1)
  it = jnp.concatenate([ie_ref[...], i1_ref[...], i2_ref[...]], axis=1)
  ug_ref[...] = jnp.where(keep, ug, 0.0)
  it_ref[...] = jnp.where(keep, it, 0.0)


def _assemble_tc(ue, soc1, soc2, sim1, sim2, ie, is1, is2):
  blk = 2048
  grid = _NPAD // blk
  spec_d = pl.BlockSpec((blk, _D), lambda i: (i, 0))
  spec_a = pl.BlockSpec((blk, _ALL_D), lambda i: (i, 0))
  return pl.pallas_call(
      _assemble_body,
      grid=(grid,),
      in_specs=[spec_d] * 8,
      out_specs=[spec_a, spec_a],
      out_shape=[
          jax.ShapeDtypeStruct((_NPAD, _ALL_D), jnp.float32),
          jax.ShapeDtypeStruct((_NPAD, _ALL_D), jnp.float32),
      ],
  )(ue, soc1, soc2, sim1, sim2, ie, is1, is2)


def _final_body(ug_ref, isg_ref, il_ref, usl_ref, femb_ref, lens_ref,
                wq_ref, wk_ref, av_ref, w1_ref, b1_ref, w2_ref, b2_ref,
                out_ref):
  blk = ug_ref.shape[0]
  usl = usl_ref[...]
  q = jnp.dot(usl, wq_ref[...], preferred_element_type=jnp.float32,
              precision=_HIGH)
  wk = wk_ref[...]
  av = av_ref[...]
  e_cols = []
  for f in range(_NF):
    kf = femb_ref[:, f, :]
    kp = jnp.dot(kf, wk, preferred_element_type=jnp.float32, precision=_HIGH)
    ef = jnp.dot(jnp.tanh(q + kp), av, preferred_element_type=jnp.float32,
                 precision=_HIGH)
    e_cols.append(ef)
  e = jnp.concatenate(e_cols, axis=1)
  lens = jnp.maximum(lens_ref[...], 1)
  mask = jax.lax.broadcasted_iota(jnp.int32, (blk, _NF), 1) < lens
  e = jnp.where(mask, e, -1e9)
  m = jnp.max(e, axis=1, keepdims=True)
  ex = jnp.exp(e - m)
  a = ex / jnp.sum(ex, axis=1, keepdims=True)
  ufl = jnp.zeros((blk, _ALL_D), jnp.float32)
  for f in range(_NF):
    ufl = ufl + a[:, f:f + 1] * femb_ref[:, f, :]
  ul = 0.5 * (usl + ufl)

  def predict(u, i, g):
    h = (jnp.dot(u, w1_ref[g, :_ALL_D, :], preferred_element_type=jnp.float32,
                 precision=_HIGH)
         + jnp.dot(i, w1_ref[g, _ALL_D:, :], preferred_element_type=jnp.float32,
                   precision=_HIGH)
         + b1_ref[g:g + 1, :])
    h = _leaky(h)
    return (jnp.dot(h, w2_ref[g], preferred_element_type=jnp.float32,
                    precision=_HIGH) + b2_ref[g:g + 1, :])

  ug = ug_ref[...]
  isg = isg_ref[...]
  s1 = predict(ug, isg, 0)
  s2 = predict(ul, isg, 1)
  s3 = predict(ug, il_ref[...], 2)
  out_ref[...] = (s1 + s2 + s3) * (1.0 / 3.0)


def _final_tc(ug, isg, il, usl, femb, lens2, att_wq, att_wk, att_v2,
              pred_w1, pred_b1, pred_w2, pred_b2):
  blk = 256
  grid = _B // blk
  spec_a = pl.BlockSpec((blk, _ALL_D), lambda i: (i, 0))
  return pl.pallas_call(
      _final_body,
      grid=(grid,),
      in_specs=[
          spec_a, spec_a, spec_a, spec_a,
          pl.BlockSpec((blk, _NF, _ALL_D), lambda i: (i, 0, 0)),
          pl.BlockSpec((blk, 1), lambda i: (i, 0)),
          pl.BlockSpec((_ALL_D, _ATT), lambda i: (0, 0)),
          pl.BlockSpec((_ALL_D, _ATT), lambda i: (0, 0)),
          pl.BlockSpec((_ATT, 1), lambda i: (0, 0)),
          pl.BlockSpec((3, 2 * _ALL_D, _SDIM), lambda i: (0, 0, 0)),
          pl.BlockSpec((3, _SDIM), lambda i: (0, 0)),
          pl.BlockSpec((3, _SDIM, 1), lambda i: (0, 0, 0)),
          pl.BlockSpec((3, 1), lambda i: (0, 0)),
      ],
      out_specs=pl.BlockSpec((blk, 1), lambda i: (i, 0)),
      out_shape=jax.ShapeDtypeStruct((_B, 1), jnp.float32),
  )(ug, isg, il, usl, femb, lens2, att_wq, att_wk, att_v2,
    pred_w1, pred_b1, pred_w2, pred_b2)


def _run_graph(x0, src, dst, ws, wn, zeros_zb, ones_ob):
  segsum = _make_segsum(_NPAD, _D, _E)
  degree = _make_degree(_NPAD, _D, _E)
  degp = degree(dst, zeros_zb, ones_ob)
  parts = segsum(x0, src, dst, zeros_zb)
  h1, n1 = _layer_tc(x0, parts, degp, ws[0], wn[0])
  parts2 = segsum(h1, src, dst, zeros_zb)
  _, n2 = _layer_tc(h1, parts2, degp, ws[1], wn[1])
  return n1, n2


def kernel(user, candidate, user_soc_edge_index, user_sim_edge_index,
           item_sim_edge_index, item_users, user_items, friends_items,
           friends_lens, user_embedding, item_embedding, gnn_w_self,
           gnn_w_neigh, att_wq, att_wk, att_v, pred_w1, pred_b1, pred_w2,
           pred_b2):
  f32 = jnp.float32
  i32 = jnp.int32
  n_u = user_embedding.shape[0]
  n_i = item_embedding.shape[0]

  ue = jnp.pad(user_embedding.astype(f32), ((0, _NPAD - n_u), (0, 0)))
  ie = jnp.pad(item_embedding.astype(f32), ((0, _NPAD - n_i), (0, 0)))
  zeros_zb = jnp.zeros((_NPAD // NSUB, _D), f32)
  ones_ob = jnp.ones((80, _D), f32)

  def _epad(v):
    return jnp.pad(v.astype(i32), (0, 160))

  soc_src = _epad(user_soc_edge_index[0])
  soc_dst = _epad(user_soc_edge_index[1])
  sim_src = _epad(user_sim_edge_index[0])
  sim_dst = _epad(user_sim_edge_index[1])
  isim_src = _epad(item_sim_edge_index[0])
  isim_dst = _epad(item_sim_edge_index[1])

  soc1, soc2 = _run_graph(ue, soc_src, soc_dst, gnn_w_self[0],
                          gnn_w_neigh[0], zeros_zb, ones_ob)
  sim1, sim2 = _run_graph(ue, sim_src, sim_dst, gnn_w_self[1],
                          gnn_w_neigh[1], zeros_zb, ones_ob)
  is1, is2 = _run_graph(ie, isim_src, isim_dst, gnn_w_self[2],
                        gnn_w_neigh[2], zeros_zb, ones_ob)

  ug_table, it_table = _assemble_tc(ue, soc1, soc2, sim1, sim2, ie, is1, is2)

  fuse1 = _make_fuse(_NPAD, _ALL_D, _B, 1, _HIST, True)
  fuse_f = _make_fuse(_NPAD, _ALL_D, _B, _NF, _HIST, False)

  item_local, user_global = fuse1(ug_table, user.astype(i32),
                                  item_users.reshape(-1).astype(i32))
  user_sim_local, item_sim_global = fuse1(it_table, candidate.astype(i32),
                                          user_items.reshape(-1).astype(i32))
  f_emb = fuse_f(it_table, candidate.astype(i32),
                 friends_items.reshape(-1).astype(i32))

  femb = f_emb.reshape(_B, _NF, _ALL_D)
  lens2 = friends_lens.astype(i32).reshape(_B, 1)
  out = _final_tc(user_global, item_sim_global, item_local, user_sim_local,
                  femb, lens2, att_wq, att_wk, att_v.reshape(_ATT, 1),
                  pred_w1, pred_b1, pred_w2, pred_b2)
  return out.reshape(_B)


# speedup vs baseline: 4.7458x; 1.0133x over previous
import functools

import jax
import jax.numpy as jnp
from jax import lax
from jax.experimental import pallas as pl
from jax.experimental.pallas import tpu as pltpu
from jax.experimental.pallas import tpu_sc as plsc

NCORE = 2
NSUB = 16
NWORKER = NCORE * NSUB

_D = 128
_ALL_D = 384
_NPAD = 10240
_E = 320000
_B = 1024
_HIST = 20
_NF = 10
_ATT = 64
_SDIM = 48

_HIGH = jax.lax.Precision.HIGHEST


def _leaky(x):
  return jnp.where(x >= 0, x, 0.01 * x)


@functools.lru_cache(maxsize=None)
def _make_segsum(n_pad, d, e):
  e_w = e // NWORKER
  ch = 128
  n_ch = e_w // ch
  tail = e_w - n_ch * ch
  n2 = (n_ch - 2) // 2
  rows_tile = n_pad // NSUB

  mesh = plsc.VectorSubcoreMesh(core_axis_name="c", subcore_axis_name="s")
  out_type = jax.ShapeDtypeStruct((NCORE, n_pad, d), jnp.float32)
  scratch = [
      pltpu.VMEM((ch,), jnp.int32),
      pltpu.VMEM((ch,), jnp.int32),
      pltpu.VMEM((ch,), jnp.int32),
      pltpu.VMEM((ch,), jnp.int32),
      pltpu.VMEM((ch, d), jnp.float32),
      pltpu.VMEM((ch, d), jnp.float32),
      pltpu.VMEM((tail,), jnp.int32),
      pltpu.VMEM((tail,), jnp.int32),
      pltpu.VMEM((tail, d), jnp.float32),
      pltpu.VMEM_SHARED((n_pad, d), jnp.float32),
      pltpu.SemaphoreType.DMA,
      pltpu.SemaphoreType.DMA,
      pltpu.SemaphoreType.DMA,
      pltpu.SemaphoreType.DMA,
      pltpu.SemaphoreType.DMA,
      pltpu.SemaphoreType.DMA,
  ]

  def body(x_hbm, src_hbm, dst_hbm, z_hbm, *refs):
    (out_hbm, is0, is1, id0, id1, r0, r1, ts_v, td_v, tr_v, acc_sh,
     si0, si1, sg0, sg1, ss0, ss1) = refs
    c = lax.axis_index("c")
    s = lax.axis_index("s")
    w = s * NCORE + c
    tb = s * rows_tile

    pltpu.sync_copy(z_hbm, acc_sh.at[pl.ds(tb, rows_tile), :])
    plsc.subcore_barrier()

    base = w * e_w

    def fire_idx(ibs, ibd, sem, chunk):
      off = base + chunk * ch
      pltpu.async_copy(src_hbm.at[pl.ds(off, ch)], ibs, sem)
      pltpu.async_copy(dst_hbm.at[pl.ds(off, ch)], ibd, sem)

    def wait_idx(ibs, ibd, sem):
      pltpu.make_async_copy(src_hbm.at[pl.ds(0, ch)], ibs, sem).wait()
      pltpu.make_async_copy(dst_hbm.at[pl.ds(0, ch)], ibd, sem).wait()

    def wait_sc(rows, ibd, sem):
      pltpu.make_async_copy(rows, acc_sh.at[ibd], sem).wait()

    toff = base + n_ch * ch
    pltpu.sync_copy(src_hbm.at[pl.ds(toff, tail)], ts_v)
    pltpu.sync_copy(dst_hbm.at[pl.ds(toff, tail)], td_v)
    pltpu.async_copy(x_hbm.at[ts_v], tr_v, sg0).wait()
    pltpu.sync_copy(tr_v, acc_sh.at[td_v], add=True)

    fire_idx(is0, id0, si0, 0)
    fire_idx(is1, id1, si1, 1)
    wait_idx(is0, id0, si0)
    pltpu.async_copy(x_hbm.at[is0], r0, sg0)
    wait_idx(is1, id1, si1)
    pltpu.async_copy(x_hbm.at[is1], r1, sg1)

    def step(i, carry):
      c0 = 2 * i
      pltpu.make_async_copy(x_hbm.at[is0], r0, sg0).wait()
      pltpu.async_copy(r0, acc_sh.at[id0], sem=ss0, add=True)
      pltpu.make_async_copy(x_hbm.at[is1], r1, sg1).wait()
      pltpu.async_copy(r1, acc_sh.at[id1], sem=ss1, add=True)
      wait_sc(r0, id0, ss0)
      fire_idx(is0, id0, si0, c0 + 2)
      wait_idx(is0, id0, si0)
      pltpu.async_copy(x_hbm.at[is0], r0, sg0)
      wait_sc(r1, id1, ss1)
      fire_idx(is1, id1, si1, c0 + 3)
      wait_idx(is1, id1, si1)
      pltpu.async_copy(x_hbm.at[is1], r1, sg1)
      return carry

    lax.fori_loop(0, n2, step, 0)
    pltpu.make_async_copy(x_hbm.at[is0], r0, sg0).wait()
    pltpu.async_copy(r0, acc_sh.at[id0], sem=ss0, add=True)
    pltpu.make_async_copy(x_hbm.at[is1], r1, sg1).wait()
    pltpu.async_copy(r1, acc_sh.at[id1], sem=ss1, add=True)
    wait_sc(r0, id0, ss0)
    wait_sc(r1, id1, ss1)

    plsc.subcore_barrier()
    pltpu.sync_copy(acc_sh.at[pl.ds(tb, rows_tile), :],
                    out_hbm.at[c, pl.ds(tb, rows_tile), :])

  return pl.kernel(body, out_type=out_type, mesh=mesh, scratch_types=scratch)


@functools.lru_cache(maxsize=None)
def _make_degree(n_pad, d, e):
  e_w = e // NWORKER
  ch = 128
  n_ch = e_w // ch
  tail = e_w - n_ch * ch
  n2 = (n_ch - 2) // 2
  rows_tile = n_pad // NSUB

  mesh = plsc.VectorSubcoreMesh(core_axis_name="c", subcore_axis_name="s")
  out_type = jax.ShapeDtypeStruct((NCORE, n_pad, d), jnp.float32)
  scratch = [
      pltpu.VMEM((ch,), jnp.int32),
      pltpu.VMEM((ch,), jnp.int32),
      pltpu.VMEM((tail,), jnp.int32),
      pltpu.VMEM((ch, d), jnp.float32),
      pltpu.VMEM_SHARED((n_pad, d), jnp.float32),
      pltpu.SemaphoreType.DMA,
      pltpu.SemaphoreType.DMA,
      pltpu.SemaphoreType.DMA,
      pltpu.SemaphoreType.DMA,
  ]

  def body(dst_hbm, z_hbm, ones_hbm, *refs):
    (deg_hbm, id0, id1, td_v, ones_v, dacc_sh, si0, si1, ss0, ss1) = refs
    c = lax.axis_index("c")
    s = lax.axis_index("s")
    w = s * NCORE + c
    tb = s * rows_tile

    pltpu.sync_copy(z_hbm, dacc_sh.at[pl.ds(tb, rows_tile), :])
    pltpu.sync_copy(ones_hbm, ones_v)
    plsc.subcore_barrier()

    base = w * e_w

    def fire(ibd, sem, chunk):
      pltpu.async_copy(dst_hbm.at[pl.ds(base + chunk * ch, ch)], ibd, sem)

    def wait(ibd, sem):
      pltpu.make_async_copy(dst_hbm.at[pl.ds(0, ch)], ibd, sem).wait()

    def wait_sc(ibd, sem):
      pltpu.make_async_copy(ones_v, dacc_sh.at[ibd], sem).wait()

    toff = base + n_ch * ch
    pltpu.sync_copy(dst_hbm.at[pl.ds(toff, tail)], td_v)
    pltpu.sync_copy(ones_v.at[pl.ds(0, tail), :], dacc_sh.at[td_v], add=True)

    fire(id0, si0, 0)
    fire(id1, si1, 1)

    def step(i, carry):
      c0 = 2 * i
      wait(id0, si0)
      pltpu.async_copy(ones_v, dacc_sh.at[id0], sem=ss0, add=True)
      wait(id1, si1)
      pltpu.async_copy(ones_v, dacc_sh.at[id1], sem=ss1, add=True)
      wait_sc(id0, ss0)
      fire(id0, si0, c0 + 2)
      wait_sc(id1, ss1)
      fire(id1, si1, c0 + 3)
      return carry

    lax.fori_loop(0, n2, step, 0)
    wait(id0, si0)
    pltpu.async_copy(ones_v, dacc_sh.at[id0], sem=ss0, add=True)
    wait(id1, si1)
    pltpu.async_copy(ones_v, dacc_sh.at[id1], sem=ss1, add=True)
    wait_sc(id0, ss0)
    wait_sc(id1, ss1)

    plsc.subcore_barrier()
    pltpu.sync_copy(dacc_sh.at[pl.ds(tb, rows_tile), :],
                    deg_hbm.at[c, pl.ds(tb, rows_tile), :])

  return pl.kernel(body, out_type=out_type, mesh=mesh, scratch_types=scratch)


@functools.lru_cache(maxsize=None)
def _make_fuse(n_pad, d_all, b, f, hist, emit_scale):
  nb = b // NWORKER
  pairs = nb * f
  pc = 4
  gi = pc * hist
  oc = 32 if f == 1 else 80
  cpw = oc // pc
  n_out = pairs // oc
  nv = d_all // 16

  mesh = plsc.VectorSubcoreMesh(core_axis_name="c", subcore_axis_name="s")
  if emit_scale:
    out_type = [jax.ShapeDtypeStruct((b * f, d_all), jnp.float32),
                jax.ShapeDtypeStruct((b, d_all), jnp.float32)]
  else:
    out_type = jax.ShapeDtypeStruct((b * f, d_all), jnp.float32)
  scratch = [
      pltpu.VMEM((nb,), jnp.int32),
      pltpu.VMEM((nb, d_all), jnp.float32),
      pltpu.VMEM((oc * hist,), jnp.int32),
      pltpu.VMEM((gi, d_all), jnp.float32),
      pltpu.VMEM((gi, d_all), jnp.float32),
      pltpu.VMEM((oc, d_all), jnp.float32),
      pltpu.SemaphoreType.DMA,
      pltpu.SemaphoreType.DMA,
      pltpu.SemaphoreType.DMA,
  ]
  c2 = cpw // 2

  def body(table_hbm, sidx_hbm, hidx_hbm, *refs):
    if emit_scale:
      (fused_hbm, scale_hbm, sidx_v, scale_v, hidx_v, r0, r1, out_v,
       sem, sg0, sg1) = refs
    else:
      (fused_hbm, sidx_v, scale_v, hidx_v, r0, r1, out_v,
       sem, sg0, sg1) = refs
      scale_hbm = None
    c = lax.axis_index("c")
    s = lax.axis_index("s")
    w = s * NCORE + c

    pltpu.sync_copy(sidx_hbm.at[pl.ds(w * nb, nb)], sidx_v)
    pltpu.async_copy(table_hbm.at[sidx_v], scale_v, sem).wait()
    if emit_scale:
      pltpu.sync_copy(scale_v, scale_hbm.at[pl.ds(w * nb, nb), :])

    def fire_g(rows, sg, t):
      tc = jnp.minimum(t, cpw - 1)
      pltpu.async_copy(table_hbm.at[hidx_v.at[pl.ds(tc * gi, gi)]], rows, sg)

    def wait_g(rows, sg):
      pltpu.make_async_copy(table_hbm.at[hidx_v.at[pl.ds(0, gi)]],
                            rows, sg).wait()

    def compute_chunk(rows, t, p_base):
      for j in range(pc):
        bloc = (p_base + j) // f
        for jv in range(nv):
          sl = pl.ds(jv * 16, 16)
          sc_vec = scale_v[bloc, sl]
          acc = rows[j * hist, sl] * sc_vec
          for h in range(1, hist):
            acc = jnp.maximum(acc, rows[j * hist + h, sl] * sc_vec)
          out_v[t * pc + j, sl] = acc

    def outer(o, carry):
      pltpu.sync_copy(hidx_hbm.at[pl.ds((w * pairs + o * oc) * hist,
                                        oc * hist)], hidx_v)
      fire_g(r0, sg0, 0)
      fire_g(r1, sg1, 1)

      def inner(i, carry2):
        ta = 2 * i
        p_blk = o * oc
        wait_g(r0, sg0)
        compute_chunk(r0, ta, p_blk + ta * pc)
        fire_g(r0, sg0, ta + 2)
        wait_g(r1, sg1)
        compute_chunk(r1, ta + 1, p_blk + (ta + 1) * pc)
        fire_g(r1, sg1, ta + 3)
        return carry2

      lax.fori_loop(0, c2, inner, 0)
      wait_g(r0, sg0)
      wait_g(r1, sg1)
      pltpu.sync_copy(out_v, fused_hbm.at[pl.ds(w * pairs + o * oc, oc), :])
      return carry

    lax.fori_loop(0, n_out, outer, 0)

  return pl.kernel(body, out_type=out_type, mesh=mesh, scratch_types=scratch)


def _layer_body(x_ref, p_ref, dp_ref, ws_ref, wn_ref, h_ref, hn_ref):
  x = x_ref[...]
  agg = p_ref[0] + p_ref[1]
  deg = dp_ref[0, :, 0:1] + dp_ref[1, :, 0:1]
  mean = agg / jnp.maximum(deg, 1.0)
  h = (jnp.dot(x, ws_ref[...], preferred_element_type=jnp.float32,
               precision=_HIGH)
       + jnp.dot(mean, wn_ref[...], preferred_element_type=jnp.float32,
                 precision=_HIGH))
  h = _leaky(h)
  nrm = jnp.sqrt(jnp.sum(h * h, axis=1, keepdims=True))
  hn = h / jnp.maximum(nrm, 1e-12)
  h_ref[...] = h
  hn_ref[...] = hn


def _layer_tc(x, parts, degp, ws, wn):
  blk = 2048
  grid = _NPAD // blk
  return pl.pallas_call(
      _layer_body,
      grid=(grid,),
      in_specs=[
          pl.BlockSpec((blk, _D), lambda i: (i, 0)),
          pl.BlockSpec((NCORE, blk, _D), lambda i: (0, i, 0)),
          pl.BlockSpec((NCORE, blk, _D), lambda i: (0, i, 0)),
          pl.BlockSpec((_D, _D), lambda i: (0, 0)),
          pl.BlockSpec((_D, _D), lambda i: (0, 0)),
      ],
      out_specs=[
          pl.BlockSpec((blk, _D), lambda i: (i, 0)),
          pl.BlockSpec((blk, _D), lambda i: (i, 0)),
      ],
      out_shape=[
          jax.ShapeDtypeStruct((_NPAD, _D), jnp.float32),
          jax.ShapeDtypeStruct((_NPAD, _D), jnp.float32),
      ],
  )(x, parts, degp, ws, wn)


def _assemble_body(ue_ref, s1_ref, s2_ref, m1_ref, m2_ref,
                   ie_ref, i1_ref, i2_ref, ug_ref, it_ref):
  i = pl.program_id(0)
  blk = ue_ref.shape[0]
  rows = jax.lax.broadcasted_iota(jnp.int32, (blk, 1), 0) + i * blk
  keep = rows != 0
  ug = jnp.concatenate(
      [ue_ref[...], 0.5 * (s1_ref[...] + m1_ref[...]),
       0.5 * (s2_ref[...] + m2_ref[...])], axis=1)
  it = jnp.concatenate([ie_ref[...], i1_ref[...], i2_ref[...]], axis=1)
  ug_ref[...] = jnp.where(keep, ug, 0.0)
  it_ref[...] = jnp.where(keep, it, 0.0)


def _assemble_tc(ue, soc1, soc2, sim1, sim2, ie, is1, is2):
  blk = 2048
  grid = _NPAD // blk
  spec_d = pl.BlockSpec((blk, _D), lambda i: (i, 0))
  spec_a = pl.BlockSpec((blk, _ALL_D), lambda i: (i, 0))
  return pl.pallas_call(
      _assemble_body,
      grid=(grid,),
      in_specs=[spec_d] * 8,
      out_specs=[spec_a, spec_a],
      out_shape=[
          jax.ShapeDtypeStruct((_NPAD, _ALL_D), jnp.float32),
          jax.ShapeDtypeStruct((_NPAD, _ALL_D), jnp.float32),
      ],
  )(ue, soc1, soc2, sim1, sim2, ie, is1, is2)


def _final_body(ug_ref, isg_ref, il_ref, usl_ref, femb_ref, lens_ref,
                wq_ref, wk_ref, av_ref, w1_ref, b1_ref, w2_ref, b2_ref,
                out_ref):
  blk = ug_ref.shape[0]
  usl = usl_ref[...]
  q = jnp.dot(usl, wq_ref[...], preferred_element_type=jnp.float32,
              precision=_HIGH)
  wk = wk_ref[...]
  av = av_ref[...]
  e_cols = []
  for f in range(_NF):
    kf = femb_ref[:, f, :]
    kp = jnp.dot(kf, wk, preferred_element_type=jnp.float32, precision=_HIGH)
    ef = jnp.dot(jnp.tanh(q + kp), av, preferred_element_type=jnp.float32,
                 precision=_HIGH)
    e_cols.append(ef)
  e = jnp.concatenate(e_cols, axis=1)
  lens = jnp.maximum(lens_ref[...], 1)
  mask = jax.lax.broadcasted_iota(jnp.int32, (blk, _NF), 1) < lens
  e = jnp.where(mask, e, -1e9)
  m = jnp.max(e, axis=1, keepdims=True)
  ex = jnp.exp(e - m)
  a = ex / jnp.sum(ex, axis=1, keepdims=True)
  ufl = jnp.zeros((blk, _ALL_D), jnp.float32)
  for f in range(_NF):
    ufl = ufl + a[:, f:f + 1] * femb_ref[:, f, :]
  ul = 0.5 * (usl + ufl)

  def predict(u, i, g):
    h = (jnp.dot(u, w1_ref[g, :_ALL_D, :], preferred_element_type=jnp.float32,
                 precision=_HIGH)
         + jnp.dot(i, w1_ref[g, _ALL_D:, :], preferred_element_type=jnp.float32,
                   precision=_HIGH)
         + b1_ref[g:g + 1, :])
    h = _leaky(h)
    return (jnp.dot(h, w2_ref[g], preferred_element_type=jnp.float32,
                    precision=_HIGH) + b2_ref[g:g + 1, :])

  ug = ug_ref[...]
  isg = isg_ref[...]
  s1 = predict(ug, isg, 0)
  s2 = predict(ul, isg, 1)
  s3 = predict(ug, il_ref[...], 2)
  out_ref[...] = (s1 + s2 + s3) * (1.0 / 3.0)


def _final_tc(ug, isg, il, usl, femb, lens2, att_wq, att_wk, att_v2,
              pred_w1, pred_b1, pred_w2, pred_b2):
  blk = 256
  grid = _B // blk
  spec_a = pl.BlockSpec((blk, _ALL_D), lambda i: (i, 0))
  return pl.pallas_call(
      _final_body,
      grid=(grid,),
      in_specs=[
          spec_a, spec_a, spec_a, spec_a,
          pl.BlockSpec((blk, _NF, _ALL_D), lambda i: (i, 0, 0)),
          pl.BlockSpec((blk, 1), lambda i: (i, 0)),
          pl.BlockSpec((_ALL_D, _ATT), lambda i: (0, 0)),
          pl.BlockSpec((_ALL_D, _ATT), lambda i: (0, 0)),
          pl.BlockSpec((_ATT, 1), lambda i: (0, 0)),
          pl.BlockSpec((3, 2 * _ALL_D, _SDIM), lambda i: (0, 0, 0)),
          pl.BlockSpec((3, _SDIM), lambda i: (0, 0)),
          pl.BlockSpec((3, _SDIM, 1), lambda i: (0, 0, 0)),
          pl.BlockSpec((3, 1), lambda i: (0, 0)),
      ],
      out_specs=pl.BlockSpec((blk, 1), lambda i: (i, 0)),
      out_shape=jax.ShapeDtypeStruct((_B, 1), jnp.float32),
  )(ug, isg, il, usl, femb, lens2, att_wq, att_wk, att_v2,
    pred_w1, pred_b1, pred_w2, pred_b2)


def _run_graph(x0, src, dst, ws, wn, zeros_zb, ones_ob):
  segsum = _make_segsum(_NPAD, _D, _E)
  degree = _make_degree(_NPAD, _D, _E)
  degp = degree(dst, zeros_zb, ones_ob)
  parts = segsum(x0, src, dst, zeros_zb)
  h1, n1 = _layer_tc(x0, parts, degp, ws[0], wn[0])
  parts2 = segsum(h1, src, dst, zeros_zb)
  _, n2 = _layer_tc(h1, parts2, degp, ws[1], wn[1])
  return n1, n2


def kernel(user, candidate, user_soc_edge_index, user_sim_edge_index,
           item_sim_edge_index, item_users, user_items, friends_items,
           friends_lens, user_embedding, item_embedding, gnn_w_self,
           gnn_w_neigh, att_wq, att_wk, att_v, pred_w1, pred_b1, pred_w2,
           pred_b2):
  f32 = jnp.float32
  i32 = jnp.int32
  n_u = user_embedding.shape[0]
  n_i = item_embedding.shape[0]

  ue = jnp.pad(user_embedding.astype(f32), ((0, _NPAD - n_u), (0, 0)))
  ie = jnp.pad(item_embedding.astype(f32), ((0, _NPAD - n_i), (0, 0)))
  zeros_zb = jnp.zeros((_NPAD // NSUB, _D), f32)
  ones_ob = jnp.ones((128, _D), f32)

  def _epad(v):
    return jnp.pad(v.astype(i32), (0, 160))

  soc_src = _epad(user_soc_edge_index[0])
  soc_dst = _epad(user_soc_edge_index[1])
  sim_src = _epad(user_sim_edge_index[0])
  sim_dst = _epad(user_sim_edge_index[1])
  isim_src = _epad(item_sim_edge_index[0])
  isim_dst = _epad(item_sim_edge_index[1])

  soc1, soc2 = _run_graph(ue, soc_src, soc_dst, gnn_w_self[0],
                          gnn_w_neigh[0], zeros_zb, ones_ob)
  sim1, sim2 = _run_graph(ue, sim_src, sim_dst, gnn_w_self[1],
                          gnn_w_neigh[1], zeros_zb, ones_ob)
  is1, is2 = _run_graph(ie, isim_src, isim_dst, gnn_w_self[2],
                        gnn_w_neigh[2], zeros_zb, ones_ob)

  ug_table, it_table = _assemble_tc(ue, soc1, soc2, sim1, sim2, ie, is1, is2)

  fuse1 = _make_fuse(_NPAD, _ALL_D, _B, 1, _HIST, True)
  fuse_f = _make_fuse(_NPAD, _ALL_D, _B, _NF, _HIST, False)

  item_local, user_global = fuse1(ug_table, user.astype(i32),
                                  item_users.reshape(-1).astype(i32))
  user_sim_local, item_sim_global = fuse1(it_table, candidate.astype(i32),
                                          user_items.reshape(-1).astype(i32))
  f_emb = fuse_f(it_table, candidate.astype(i32),
                 friends_items.reshape(-1).astype(i32))

  femb = f_emb.reshape(_B, _NF, _ALL_D)
  lens2 = friends_lens.astype(i32).reshape(_B, 1)
  out = _final_tc(user_global, item_sim_global, item_local, user_sim_local,
                  femb, lens2, att_wq, att_wk, att_v.reshape(_ATT, 1),
                  pred_w1, pred_b1, pred_w2, pred_b2)
  return out.reshape(_B)


# speedup vs baseline: 5.2228x; 1.1005x over previous
import functools

import jax
import jax.numpy as jnp
from jax import lax
from jax.experimental import pallas as pl
from jax.experimental.pallas import tpu as pltpu
from jax.experimental.pallas import tpu_sc as plsc

NCORE = 2
NSUB = 16
NWORKER = NCORE * NSUB

_D = 128
_ALL_D = 384
_NPAD = 10240
_E = 320000
_B = 1024
_HIST = 20
_NF = 10
_ATT = 64
_SDIM = 48

_HIGH = jax.lax.Precision.HIGHEST


def _leaky(x):
  return jnp.where(x >= 0, x, 0.01 * x)


@functools.lru_cache(maxsize=None)
def _make_segsum(n_pad, d, e):
  e_w = e // NWORKER
  ch = 128
  n_ch = e_w // ch
  tail = e_w - n_ch * ch
  n2 = (n_ch - 2) // 2
  rows_tile = n_pad // NSUB

  mesh = plsc.VectorSubcoreMesh(core_axis_name="c", subcore_axis_name="s")
  out_type = jax.ShapeDtypeStruct((NCORE, n_pad, d), jnp.float32)
  scratch = [
      pltpu.VMEM((ch,), jnp.int32),
      pltpu.VMEM((ch,), jnp.int32),
      pltpu.VMEM((ch,), jnp.int32),
      pltpu.VMEM((ch,), jnp.int32),
      pltpu.VMEM((ch, d), jnp.float32),
      pltpu.VMEM((ch, d), jnp.float32),
      pltpu.VMEM((tail,), jnp.int32),
      pltpu.VMEM((tail,), jnp.int32),
      pltpu.VMEM((tail, d), jnp.float32),
      pltpu.VMEM_SHARED((n_pad, d), jnp.float32),
      pltpu.SemaphoreType.DMA,
      pltpu.SemaphoreType.DMA,
      pltpu.SemaphoreType.DMA,
      pltpu.SemaphoreType.DMA,
      pltpu.SemaphoreType.DMA,
      pltpu.SemaphoreType.DMA,
  ]

  def body(x_hbm, src_hbm, dst_hbm, z_hbm, *refs):
    (out_hbm, is0, is1, id0, id1, r0, r1, ts_v, td_v, tr_v, acc_sh,
     si0, si1, sg0, sg1, ss0, ss1) = refs
    c = lax.axis_index("c")
    s = lax.axis_index("s")
    w = s * NCORE + c
    tb = s * rows_tile

    pltpu.sync_copy(z_hbm, acc_sh.at[pl.ds(tb, rows_tile), :])
    plsc.subcore_barrier()

    base = w * e_w

    def fire_idx(ibs, ibd, sem, chunk):
      off = base + chunk * ch
      pltpu.async_copy(src_hbm.at[pl.ds(off, ch)], ibs, sem)
      pltpu.async_copy(dst_hbm.at[pl.ds(off, ch)], ibd, sem)

    def wait_idx(ibs, ibd, sem):
      pltpu.make_async_copy(src_hbm.at[pl.ds(0, ch)], ibs, sem).wait()
      pltpu.make_async_copy(dst_hbm.at[pl.ds(0, ch)], ibd, sem).wait()

    def wait_sc(rows, ibd, sem):
      pltpu.make_async_copy(rows, acc_sh.at[ibd], sem).wait()

    toff = base + n_ch * ch
    pltpu.sync_copy(src_hbm.at[pl.ds(toff, tail)], ts_v)
    pltpu.sync_copy(dst_hbm.at[pl.ds(toff, tail)], td_v)
    pltpu.async_copy(x_hbm.at[ts_v], tr_v, sg0).wait()
    pltpu.sync_copy(tr_v, acc_sh.at[td_v], add=True)

    fire_idx(is0, id0, si0, 0)
    fire_idx(is1, id1, si1, 1)
    wait_idx(is0, id0, si0)
    pltpu.async_copy(x_hbm.at[is0], r0, sg0)
    wait_idx(is1, id1, si1)
    pltpu.async_copy(x_hbm.at[is1], r1, sg1)

    def step(i, carry):
      c0 = 2 * i
      pltpu.make_async_copy(x_hbm.at[is0], r0, sg0).wait()
      pltpu.async_copy(r0, acc_sh.at[id0], sem=ss0, add=True)
      pltpu.make_async_copy(x_hbm.at[is1], r1, sg1).wait()
      pltpu.async_copy(r1, acc_sh.at[id1], sem=ss1, add=True)
      wait_sc(r0, id0, ss0)
      fire_idx(is0, id0, si0, c0 + 2)
      wait_idx(is0, id0, si0)
      pltpu.async_copy(x_hbm.at[is0], r0, sg0)
      wait_sc(r1, id1, ss1)
      fire_idx(is1, id1, si1, c0 + 3)
      wait_idx(is1, id1, si1)
      pltpu.async_copy(x_hbm.at[is1], r1, sg1)
      return carry

    lax.fori_loop(0, n2, step, 0)
    pltpu.make_async_copy(x_hbm.at[is0], r0, sg0).wait()
    pltpu.async_copy(r0, acc_sh.at[id0], sem=ss0, add=True)
    pltpu.make_async_copy(x_hbm.at[is1], r1, sg1).wait()
    pltpu.async_copy(r1, acc_sh.at[id1], sem=ss1, add=True)
    wait_sc(r0, id0, ss0)
    wait_sc(r1, id1, ss1)

    plsc.subcore_barrier()
    pltpu.sync_copy(acc_sh.at[pl.ds(tb, rows_tile), :],
                    out_hbm.at[c, pl.ds(tb, rows_tile), :])

  return pl.kernel(body, out_type=out_type, mesh=mesh, scratch_types=scratch)


@functools.lru_cache(maxsize=None)
def _make_degree(n_pad, d, e):
  e_w = e // NWORKER
  ch = 128
  n_ch = e_w // ch
  tail = e_w - n_ch * ch
  n2 = (n_ch - 2) // 2
  rows_tile = n_pad // NSUB

  mesh = plsc.VectorSubcoreMesh(core_axis_name="c", subcore_axis_name="s")
  out_type = jax.ShapeDtypeStruct((NCORE, n_pad, d), jnp.float32)
  scratch = [
      pltpu.VMEM((ch,), jnp.int32),
      pltpu.VMEM((ch,), jnp.int32),
      pltpu.VMEM((tail,), jnp.int32),
      pltpu.VMEM((ch, d), jnp.float32),
      pltpu.VMEM_SHARED((n_pad, d), jnp.float32),
      pltpu.SemaphoreType.DMA,
      pltpu.SemaphoreType.DMA,
      pltpu.SemaphoreType.DMA,
      pltpu.SemaphoreType.DMA,
  ]

  def body(dst_hbm, z_hbm, ones_hbm, *refs):
    (deg_hbm, id0, id1, td_v, ones_v, dacc_sh, si0, si1, ss0, ss1) = refs
    c = lax.axis_index("c")
    s = lax.axis_index("s")
    w = s * NCORE + c
    tb = s * rows_tile

    pltpu.sync_copy(z_hbm, dacc_sh.at[pl.ds(tb, rows_tile), :])
    pltpu.sync_copy(ones_hbm, ones_v)
    plsc.subcore_barrier()

    base = w * e_w

    def fire(ibd, sem, chunk):
      pltpu.async_copy(dst_hbm.at[pl.ds(base + chunk * ch, ch)], ibd, sem)

    def wait(ibd, sem):
      pltpu.make_async_copy(dst_hbm.at[pl.ds(0, ch)], ibd, sem).wait()

    def wait_sc(ibd, sem):
      pltpu.make_async_copy(ones_v, dacc_sh.at[ibd], sem).wait()

    toff = base + n_ch * ch
    pltpu.sync_copy(dst_hbm.at[pl.ds(toff, tail)], td_v)
    pltpu.sync_copy(ones_v.at[pl.ds(0, tail), :], dacc_sh.at[td_v], add=True)

    fire(id0, si0, 0)
    fire(id1, si1, 1)

    def step(i, carry):
      c0 = 2 * i
      wait(id0, si0)
      pltpu.async_copy(ones_v, dacc_sh.at[id0], sem=ss0, add=True)
      wait(id1, si1)
      pltpu.async_copy(ones_v, dacc_sh.at[id1], sem=ss1, add=True)
      wait_sc(id0, ss0)
      fire(id0, si0, c0 + 2)
      wait_sc(id1, ss1)
      fire(id1, si1, c0 + 3)
      return carry

    lax.fori_loop(0, n2, step, 0)
    wait(id0, si0)
    pltpu.async_copy(ones_v, dacc_sh.at[id0], sem=ss0, add=True)
    wait(id1, si1)
    pltpu.async_copy(ones_v, dacc_sh.at[id1], sem=ss1, add=True)
    wait_sc(id0, ss0)
    wait_sc(id1, ss1)

    plsc.subcore_barrier()
    pltpu.sync_copy(dacc_sh.at[pl.ds(tb, rows_tile), :],
                    deg_hbm.at[c, pl.ds(tb, rows_tile), :])

  return pl.kernel(body, out_type=out_type, mesh=mesh, scratch_types=scratch)


@functools.lru_cache(maxsize=None)
def _make_degree_lanes(n_pad, e):
  e_w = e // NWORKER
  ch = 128
  n_ch = e_w // ch
  tail = e_w - n_ch * ch
  n2 = (n_ch - 2) // 2

  mesh = plsc.VectorSubcoreMesh(core_axis_name="c", subcore_axis_name="s")
  out_type = jax.ShapeDtypeStruct((NWORKER, n_pad), jnp.float32)
  scratch = [
      pltpu.VMEM((ch,), jnp.int32),
      pltpu.VMEM((ch,), jnp.int32),
      pltpu.VMEM((tail,), jnp.int32),
      pltpu.VMEM((n_pad,), jnp.float32),
      pltpu.SemaphoreType.DMA,
      pltpu.SemaphoreType.DMA,
  ]

  def body(dst_hbm, z_hbm, *refs):
    (deg_hbm, id0, id1, td_v, dacc, si0, si1) = refs
    c = lax.axis_index("c")
    s = lax.axis_index("s")
    w = s * NCORE + c

    pltpu.sync_copy(z_hbm, dacc)

    base = w * e_w
    ones16 = jnp.ones((16,), jnp.float32)

    def fire(ibd, sem, chunk):
      pltpu.async_copy(dst_hbm.at[pl.ds(base + chunk * ch, ch)], ibd, sem)

    def wait(ibd, sem):
      pltpu.make_async_copy(dst_hbm.at[pl.ds(0, ch)], ibd, sem).wait()

    def accum(ibd, n):
      for k in range(n // 16):
        idx = ibd[pl.ds(k * 16, 16)]
        plsc.addupdate_scatter(dacc, [idx], ones16)

    pltpu.sync_copy(dst_hbm.at[pl.ds(base + n_ch * ch, tail)], td_v)
    accum(td_v, tail)

    fire(id0, si0, 0)
    fire(id1, si1, 1)

    def step(i, carry):
      c0 = 2 * i
      wait(id0, si0)
      accum(id0, ch)
      fire(id0, si0, c0 + 2)
      wait(id1, si1)
      accum(id1, ch)
      fire(id1, si1, c0 + 3)
      return carry

    lax.fori_loop(0, n2, step, 0)
    wait(id0, si0)
    accum(id0, ch)
    wait(id1, si1)
    accum(id1, ch)

    pltpu.sync_copy(dacc, deg_hbm.at[w])

  return pl.kernel(
      body, out_type=out_type, mesh=mesh, scratch_types=scratch,
      compiler_params=pltpu.CompilerParams(needs_layout_passes=False))


@functools.lru_cache(maxsize=None)
def _make_fuse(n_pad, d_all, b, f, hist, emit_scale):
  nb = b // NWORKER
  pairs = nb * f
  pc = 4
  gi = pc * hist
  oc = 32 if f == 1 else 80
  cpw = oc // pc
  n_out = pairs // oc
  nv = d_all // 16

  mesh = plsc.VectorSubcoreMesh(core_axis_name="c", subcore_axis_name="s")
  if emit_scale:
    out_type = [jax.ShapeDtypeStruct((b * f, d_all), jnp.float32),
                jax.ShapeDtypeStruct((b, d_all), jnp.float32)]
  else:
    out_type = jax.ShapeDtypeStruct((b * f, d_all), jnp.float32)
  scratch = [
      pltpu.VMEM((nb,), jnp.int32),
      pltpu.VMEM((nb, d_all), jnp.float32),
      pltpu.VMEM((oc * hist,), jnp.int32),
      pltpu.VMEM((gi, d_all), jnp.float32),
      pltpu.VMEM((gi, d_all), jnp.float32),
      pltpu.VMEM((oc, d_all), jnp.float32),
      pltpu.SemaphoreType.DMA,
      pltpu.SemaphoreType.DMA,
      pltpu.SemaphoreType.DMA,
  ]
  c2 = cpw // 2

  def body(table_hbm, sidx_hbm, hidx_hbm, *refs):
    if emit_scale:
      (fused_hbm, scale_hbm, sidx_v, scale_v, hidx_v, r0, r1, out_v,
       sem, sg0, sg1) = refs
    else:
      (fused_hbm, sidx_v, scale_v, hidx_v, r0, r1, out_v,
       sem, sg0, sg1) = refs
      scale_hbm = None
    c = lax.axis_index("c")
    s = lax.axis_index("s")
    w = s * NCORE + c

    pltpu.sync_copy(sidx_hbm.at[pl.ds(w * nb, nb)], sidx_v)
    pltpu.async_copy(table_hbm.at[sidx_v], scale_v, sem).wait()
    if emit_scale:
      pltpu.sync_copy(scale_v, scale_hbm.at[pl.ds(w * nb, nb), :])

    def fire_g(rows, sg, t):
      tc = jnp.minimum(t, cpw - 1)
      pltpu.async_copy(table_hbm.at[hidx_v.at[pl.ds(tc * gi, gi)]], rows, sg)

    def wait_g(rows, sg):
      pltpu.make_async_copy(table_hbm.at[hidx_v.at[pl.ds(0, gi)]],
                            rows, sg).wait()

    def compute_chunk(rows, t, p_base):
      for j in range(pc):
        bloc = (p_base + j) // f
        for jv in range(nv):
          sl = pl.ds(jv * 16, 16)
          sc_vec = scale_v[bloc, sl]
          acc = rows[j * hist, sl] * sc_vec
          for h in range(1, hist):
            acc = jnp.maximum(acc, rows[j * hist + h, sl] * sc_vec)
          out_v[t * pc + j, sl] = acc

    def outer(o, carry):
      pltpu.sync_copy(hidx_hbm.at[pl.ds((w * pairs + o * oc) * hist,
                                        oc * hist)], hidx_v)
      fire_g(r0, sg0, 0)
      fire_g(r1, sg1, 1)

      def inner(i, carry2):
        ta = 2 * i
        p_blk = o * oc
        wait_g(r0, sg0)
        compute_chunk(r0, ta, p_blk + ta * pc)
        fire_g(r0, sg0, ta + 2)
        wait_g(r1, sg1)
        compute_chunk(r1, ta + 1, p_blk + (ta + 1) * pc)
        fire_g(r1, sg1, ta + 3)
        return carry2

      lax.fori_loop(0, c2, inner, 0)
      wait_g(r0, sg0)
      wait_g(r1, sg1)
      pltpu.sync_copy(out_v, fused_hbm.at[pl.ds(w * pairs + o * oc, oc), :])
      return carry

    lax.fori_loop(0, n_out, outer, 0)

  return pl.kernel(body, out_type=out_type, mesh=mesh, scratch_types=scratch)


def _layer_body(x_ref, p_ref, dp_ref, ws_ref, wn_ref, h_ref, hn_ref):
  x = x_ref[...]
  agg = p_ref[0] + p_ref[1]
  deg = jnp.sum(dp_ref[...], axis=1, keepdims=True)
  mean = agg / jnp.maximum(deg, 1.0)
  h = (jnp.dot(x, ws_ref[...], preferred_element_type=jnp.float32,
               precision=_HIGH)
       + jnp.dot(mean, wn_ref[...], preferred_element_type=jnp.float32,
                 precision=_HIGH))
  h = _leaky(h)
  nrm = jnp.sqrt(jnp.sum(h * h, axis=1, keepdims=True))
  hn = h / jnp.maximum(nrm, 1e-12)
  h_ref[...] = h
  hn_ref[...] = hn


def _layer_tc(x, parts, degp, ws, wn):
  blk = 2048
  grid = _NPAD // blk
  return pl.pallas_call(
      _layer_body,
      grid=(grid,),
      in_specs=[
          pl.BlockSpec((blk, _D), lambda i: (i, 0)),
          pl.BlockSpec((NCORE, blk, _D), lambda i: (0, i, 0)),
          pl.BlockSpec((blk, NWORKER), lambda i: (i, 0)),
          pl.BlockSpec((_D, _D), lambda i: (0, 0)),
          pl.BlockSpec((_D, _D), lambda i: (0, 0)),
      ],
      out_specs=[
          pl.BlockSpec((blk, _D), lambda i: (i, 0)),
          pl.BlockSpec((blk, _D), lambda i: (i, 0)),
      ],
      out_shape=[
          jax.ShapeDtypeStruct((_NPAD, _D), jnp.float32),
          jax.ShapeDtypeStruct((_NPAD, _D), jnp.float32),
      ],
  )(x, parts, degp, ws, wn)


def _assemble_body(ue_ref, s1_ref, s2_ref, m1_ref, m2_ref,
                   ie_ref, i1_ref, i2_ref, ug_ref, it_ref):
  i = pl.program_id(0)
  blk = ue_ref.shape[0]
  rows = jax.lax.broadcasted_iota(jnp.int32, (blk, 1), 0) + i * blk
  keep = rows != 0
  ug = jnp.concatenate(
      [ue_ref[...], 0.5 * (s1_ref[...] + m1_ref[...]),
       0.5 * (s2_ref[...] + m2_ref[...])], axis=1)
  it = jnp.concatenate([ie_ref[...], i1_ref[...], i2_ref[...]], axis=1)
  ug_ref[...] = jnp.where(keep, ug, 0.0)
  it_ref[...] = jnp.where(keep, it, 0.0)


def _assemble_tc(ue, soc1, soc2, sim1, sim2, ie, is1, is2):
  blk = 2048
  grid = _NPAD // blk
  spec_d = pl.BlockSpec((blk, _D), lambda i: (i, 0))
  spec_a = pl.BlockSpec((blk, _ALL_D), lambda i: (i, 0))
  return pl.pallas_call(
      _assemble_body,
      grid=(grid,),
      in_specs=[spec_d] * 8,
      out_specs=[spec_a, spec_a],
      out_shape=[
          jax.ShapeDtypeStruct((_NPAD, _ALL_D), jnp.float32),
          jax.ShapeDtypeStruct((_NPAD, _ALL_D), jnp.float32),
      ],
  )(ue, soc1, soc2, sim1, sim2, ie, is1, is2)


def _final_body(ug_ref, isg_ref, il_ref, usl_ref, femb_ref, lens_ref,
                wq_ref, wk_ref, av_ref, w1_ref, b1_ref, w2_ref, b2_ref,
                out_ref):
  blk = ug_ref.shape[0]
  usl = usl_ref[...]
  q = jnp.dot(usl, wq_ref[...], preferred_element_type=jnp.float32,
              precision=_HIGH)
  wk = wk_ref[...]
  av = av_ref[...]
  e_cols = []
  for f in range(_NF):
    kf = femb_ref[:, f, :]
    kp = jnp.dot(kf, wk, preferred_element_type=jnp.float32, precision=_HIGH)
    ef = jnp.dot(jnp.tanh(q + kp), av, preferred_element_type=jnp.float32,
                 precision=_HIGH)
    e_cols.append(ef)
  e = jnp.concatenate(e_cols, axis=1)
  lens = jnp.maximum(lens_ref[...], 1)
  mask = jax.lax.broadcasted_iota(jnp.int32, (blk, _NF), 1) < lens
  e = jnp.where(mask, e, -1e9)
  m = jnp.max(e, axis=1, keepdims=True)
  ex = jnp.exp(e - m)
  a = ex / jnp.sum(ex, axis=1, keepdims=True)
  ufl = jnp.zeros((blk, _ALL_D), jnp.float32)
  for f in range(_NF):
    ufl = ufl + a[:, f:f + 1] * femb_ref[:, f, :]
  ul = 0.5 * (usl + ufl)

  def predict(u, i, g):
    h = (jnp.dot(u, w1_ref[g, :_ALL_D, :], preferred_element_type=jnp.float32,
                 precision=_HIGH)
         + jnp.dot(i, w1_ref[g, _ALL_D:, :], preferred_element_type=jnp.float32,
                   precision=_HIGH)
         + b1_ref[g:g + 1, :])
    h = _leaky(h)
    return (jnp.dot(h, w2_ref[g], preferred_element_type=jnp.float32,
                    precision=_HIGH) + b2_ref[g:g + 1, :])

  ug = ug_ref[...]
  isg = isg_ref[...]
  s1 = predict(ug, isg, 0)
  s2 = predict(ul, isg, 1)
  s3 = predict(ug, il_ref[...], 2)
  out_ref[...] = (s1 + s2 + s3) * (1.0 / 3.0)


def _final_tc(ug, isg, il, usl, femb, lens2, att_wq, att_wk, att_v2,
              pred_w1, pred_b1, pred_w2, pred_b2):
  blk = 256
  grid = _B // blk
  spec_a = pl.BlockSpec((blk, _ALL_D), lambda i: (i, 0))
  return pl.pallas_call(
      _final_body,
      grid=(grid,),
      in_specs=[
          spec_a, spec_a, spec_a, spec_a,
          pl.BlockSpec((blk, _NF, _ALL_D), lambda i: (i, 0, 0)),
          pl.BlockSpec((blk, 1), lambda i: (i, 0)),
          pl.BlockSpec((_ALL_D, _ATT), lambda i: (0, 0)),
          pl.BlockSpec((_ALL_D, _ATT), lambda i: (0, 0)),
          pl.BlockSpec((_ATT, 1), lambda i: (0, 0)),
          pl.BlockSpec((3, 2 * _ALL_D, _SDIM), lambda i: (0, 0, 0)),
          pl.BlockSpec((3, _SDIM), lambda i: (0, 0)),
          pl.BlockSpec((3, _SDIM, 1), lambda i: (0, 0, 0)),
          pl.BlockSpec((3, 1), lambda i: (0, 0)),
      ],
      out_specs=pl.BlockSpec((blk, 1), lambda i: (i, 0)),
      out_shape=jax.ShapeDtypeStruct((_B, 1), jnp.float32),
  )(ug, isg, il, usl, femb, lens2, att_wq, att_wk, att_v2,
    pred_w1, pred_b1, pred_w2, pred_b2)


def _run_graph(x0, src, dst, ws, wn, zeros_zb, zeros_row):
  segsum = _make_segsum(_NPAD, _D, _E)
  degree = _make_degree_lanes(_NPAD, _E)
  degp = degree(dst, zeros_row).T
  parts = segsum(x0, src, dst, zeros_zb)
  h1, n1 = _layer_tc(x0, parts, degp, ws[0], wn[0])
  parts2 = segsum(h1, src, dst, zeros_zb)
  _, n2 = _layer_tc(h1, parts2, degp, ws[1], wn[1])
  return n1, n2


def kernel(user, candidate, user_soc_edge_index, user_sim_edge_index,
           item_sim_edge_index, item_users, user_items, friends_items,
           friends_lens, user_embedding, item_embedding, gnn_w_self,
           gnn_w_neigh, att_wq, att_wk, att_v, pred_w1, pred_b1, pred_w2,
           pred_b2):
  f32 = jnp.float32
  i32 = jnp.int32
  n_u = user_embedding.shape[0]
  n_i = item_embedding.shape[0]

  ue = jnp.pad(user_embedding.astype(f32), ((0, _NPAD - n_u), (0, 0)))
  ie = jnp.pad(item_embedding.astype(f32), ((0, _NPAD - n_i), (0, 0)))
  zeros_zb = jnp.zeros((_NPAD // NSUB, _D), f32)
  zeros_row = jnp.zeros((_NPAD,), f32)

  def _epad(v):
    return jnp.pad(v.astype(i32), (0, 160))

  soc_src = _epad(user_soc_edge_index[0])
  soc_dst = _epad(user_soc_edge_index[1])
  sim_src = _epad(user_sim_edge_index[0])
  sim_dst = _epad(user_sim_edge_index[1])
  isim_src = _epad(item_sim_edge_index[0])
  isim_dst = _epad(item_sim_edge_index[1])

  soc1, soc2 = _run_graph(ue, soc_src, soc_dst, gnn_w_self[0],
                          gnn_w_neigh[0], zeros_zb, zeros_row)
  sim1, sim2 = _run_graph(ue, sim_src, sim_dst, gnn_w_self[1],
                          gnn_w_neigh[1], zeros_zb, zeros_row)
  is1, is2 = _run_graph(ie, isim_src, isim_dst, gnn_w_self[2],
                        gnn_w_neigh[2], zeros_zb, zeros_row)

  ug_table, it_table = _assemble_tc(ue, soc1, soc2, sim1, sim2, ie, is1, is2)

  fuse1 = _make_fuse(_NPAD, _ALL_D, _B, 1, _HIST, True)
  fuse_f = _make_fuse(_NPAD, _ALL_D, _B, _NF, _HIST, False)

  item_local, user_global = fuse1(ug_table, user.astype(i32),
                                  item_users.reshape(-1).astype(i32))
  user_sim_local, item_sim_global = fuse1(it_table, candidate.astype(i32),
                                          user_items.reshape(-1).astype(i32))
  f_emb = fuse_f(it_table, candidate.astype(i32),
                 friends_items.reshape(-1).astype(i32))

  femb = f_emb.reshape(_B, _NF, _ALL_D)
  lens2 = friends_lens.astype(i32).reshape(_B, 1)
  out = _final_tc(user_global, item_sim_global, item_local, user_sim_local,
                  femb, lens2, att_wq, att_wk, att_v.reshape(_ATT, 1),
                  pred_w1, pred_b1, pred_w2, pred_b2)
  return out.reshape(_B)
